# initial kernel scaffold (unmeasured)
import jax
import jax.numpy as jnp
from jax import lax
from jax.experimental import pallas as pl
from jax.experimental.pallas import tpu as pltpu

N_DEV = 4
EPS = 1e-5


def _partial_stats(x):
    M, Nl = x.shape
    RB = 1024
    nblk = M // RB
    groups = Nl // 128

    def body(x_ref, s_ref, q_ref):
        xb = x_ref[:, :].astype(jnp.float32)
        x3 = xb.reshape(RB, groups, 128)
        s_ref[:, :] = jnp.sum(x3, axis=1)
        q_ref[:, :] = jnp.sum(x3 * x3, axis=1)

    return pl.pallas_call(
        body,
        grid=(nblk,),
        in_specs=[pl.BlockSpec((RB, Nl), lambda b: (b, 0))],
        out_specs=[
            pl.BlockSpec((RB, 128), lambda b: (b, 0)),
            pl.BlockSpec((RB, 128), lambda b: (b, 0)),
        ],
        out_shape=[
            jax.ShapeDtypeStruct((M, 128), jnp.float32),
            jax.ShapeDtypeStruct((M, 128), jnp.float32),
        ],
    )(x)


def _allreduce_stats(stats):

    def body(st_ref, out_ref, comm_ref, send_sems, recv_sems):
        my = lax.axis_index("i")

        sends = []
        for off in (1, 2, 3):
            tgt = lax.rem(my + off, N_DEV)
            r = pltpu.make_async_remote_copy(
                src_ref=st_ref,
                dst_ref=comm_ref.at[my],
                send_sem=send_sems.at[off],
                recv_sem=recv_sems.at[my],
                device_id=(tgt,),
                device_id_type=pl.DeviceIdType.MESH,
            )
            r.start()
            sends.append(r)

        acc = st_ref[:, :]
        for off in (1, 2, 3):
            src = lax.rem(my - off + N_DEV, N_DEV)
            recv = pltpu.make_async_remote_copy(
                src_ref=st_ref,
                dst_ref=comm_ref.at[src],
                send_sem=send_sems.at[0],
                recv_sem=recv_sems.at[src],
                device_id=(my,),
                device_id_type=pl.DeviceIdType.MESH,
            )
            recv.wait_recv()
            acc = acc + comm_ref[src]
        out_ref[:, :] = acc

        for r in sends:
            r.wait_send()

    return pl.pallas_call(
        body,
        out_shape=jax.ShapeDtypeStruct((128, 128), jnp.float32),
        in_specs=[pl.BlockSpec(memory_space=pltpu.VMEM)],
        out_specs=pl.BlockSpec(memory_space=pltpu.VMEM),
        scratch_shapes=[
            pltpu.VMEM((N_DEV, 128, 128), jnp.float32),
            pltpu.SemaphoreType.DMA((N_DEV,)),
            pltpu.SemaphoreType.DMA((N_DEV,)),
        ],
        compiler_params=pltpu.CompilerParams(collective_id=0),
    )(stats)


def _normalize(x, mean, rstd, gamma2, beta2):
    M, Nl = x.shape
    RB = 1024
    nblk = M // RB

    def body(x_ref, m_ref, r_ref, g_ref, b_ref, o_ref):
        xb = x_ref[:, :].astype(jnp.float32)
        o_ref[:, :] = (xb - m_ref[:, :]) * r_ref[:, :] * g_ref[:, :] + b_ref[:, :]

    return pl.pallas_call(
        body,
        grid=(nblk,),
        in_specs=[
            pl.BlockSpec((RB, Nl), lambda b: (b, 0)),
            pl.BlockSpec((RB, 1), lambda b: (b, 0)),
            pl.BlockSpec((RB, 1), lambda b: (b, 0)),
            pl.BlockSpec((1, Nl), lambda b: (0, 0)),
            pl.BlockSpec((1, Nl), lambda b: (0, 0)),
        ],
        out_specs=pl.BlockSpec((RB, Nl), lambda b: (b, 0)),
        out_shape=jax.ShapeDtypeStruct((M, Nl), jnp.float32),
    )(x, mean, rstd, gamma2, beta2)


def kernel(x, gamma, beta):
    M, Nl = x.shape
    n_global = Nl * N_DEV

    s16, q16 = _partial_stats(x)
    sums = jnp.sum(s16, axis=1)
    sumsq = jnp.sum(q16, axis=1)
    stats = jnp.concatenate([sums[:, None], sumsq[:, None]], axis=1)
    stats = stats.reshape(128, 128)

    red = _allreduce_stats(stats)

    tot = red.reshape(M, 2)
    mean = tot[:, 0:1] / n_global
    var = tot[:, 1:2] / n_global - mean * mean
    rstd = lax.rsqrt(var + EPS)

    return _normalize(
        x, mean, rstd, gamma.reshape(1, Nl), beta.reshape(1, Nl)
    )


# baseline (device time: 98376 ns/iter reference)
import jax
import jax.numpy as jnp
from jax import lax
from jax.experimental import pallas as pl
from jax.experimental.pallas import tpu as pltpu

N_DEV = 4
EPS = 1e-5


def _partial_stats(x):
    M, Nl = x.shape
    RB = 1024
    nblk = M // RB
    groups = Nl // 128

    def body(x_ref, s_ref, q_ref):
        xb = x_ref[:, :].astype(jnp.float32)
        x3 = xb.reshape(RB, groups, 128)
        s_ref[:, :] = jnp.sum(x3, axis=1)
        q_ref[:, :] = jnp.sum(x3 * x3, axis=1)

    return pl.pallas_call(
        body,
        grid=(nblk,),
        in_specs=[pl.BlockSpec((RB, Nl), lambda b: (b, 0))],
        out_specs=[
            pl.BlockSpec((RB, 128), lambda b: (b, 0)),
            pl.BlockSpec((RB, 128), lambda b: (b, 0)),
        ],
        out_shape=[
            jax.ShapeDtypeStruct((M, 128), jnp.float32),
            jax.ShapeDtypeStruct((M, 128), jnp.float32),
        ],
    )(x)


def _allreduce_stats(stats):

    def body(st_ref, out_ref, comm_ref, send_sems, recv_sems):
        my = lax.axis_index("i")

        sends = []
        for off in (1, 2, 3):
            tgt = lax.rem(my + off, N_DEV)
            r = pltpu.make_async_remote_copy(
                src_ref=st_ref,
                dst_ref=comm_ref.at[my],
                send_sem=send_sems.at[off],
                recv_sem=recv_sems.at[my],
                device_id=(tgt,),
                device_id_type=pl.DeviceIdType.MESH,
            )
            r.start()
            sends.append(r)

        acc = st_ref[:, :]
        for off in (1, 2, 3):
            src = lax.rem(my - off + N_DEV, N_DEV)
            recv = pltpu.make_async_remote_copy(
                src_ref=st_ref,
                dst_ref=comm_ref.at[src],
                send_sem=send_sems.at[0],
                recv_sem=recv_sems.at[src],
                device_id=(my,),
                device_id_type=pl.DeviceIdType.MESH,
            )
            recv.wait_recv()
            acc = acc + comm_ref[src]
        out_ref[:, :] = acc

        for r in sends:
            r.wait_send()

    return pl.pallas_call(
        body,
        out_shape=jax.ShapeDtypeStruct((128, 128), jnp.float32),
        in_specs=[pl.BlockSpec(memory_space=pltpu.VMEM)],
        out_specs=pl.BlockSpec(memory_space=pltpu.VMEM),
        scratch_shapes=[
            pltpu.VMEM((N_DEV, 128, 128), jnp.float32),
            pltpu.SemaphoreType.DMA((N_DEV,)),
            pltpu.SemaphoreType.DMA((N_DEV,)),
        ],
    )(stats)


def _normalize(x, mean, rstd, gamma2, beta2):
    M, Nl = x.shape
    RB = 512
    nblk = M // RB

    def body(x_ref, m_ref, r_ref, g_ref, b_ref, o_ref):
        xb = x_ref[:, :].astype(jnp.float32)
        o_ref[:, :] = (xb - m_ref[:, :]) * r_ref[:, :] * g_ref[:, :] + b_ref[:, :]

    return pl.pallas_call(
        body,
        grid=(nblk,),
        in_specs=[
            pl.BlockSpec((RB, Nl), lambda b: (b, 0)),
            pl.BlockSpec((RB, 1), lambda b: (b, 0)),
            pl.BlockSpec((RB, 1), lambda b: (b, 0)),
            pl.BlockSpec((1, Nl), lambda b: (0, 0)),
            pl.BlockSpec((1, Nl), lambda b: (0, 0)),
        ],
        out_specs=pl.BlockSpec((RB, Nl), lambda b: (b, 0)),
        out_shape=jax.ShapeDtypeStruct((M, Nl), jnp.float32),
    )(x, mean, rstd, gamma2, beta2)


def kernel(x, gamma, beta):
    M, Nl = x.shape
    n_global = Nl * N_DEV

    s16, q16 = _partial_stats(x)
    sums = jnp.sum(s16, axis=1)
    sumsq = jnp.sum(q16, axis=1)
    stats = jnp.concatenate([sums[:, None], sumsq[:, None]], axis=1)
    stats = stats.reshape(128, 128)

    red = _allreduce_stats(stats)

    tot = red.reshape(M, 2)
    mean = tot[:, 0:1] / n_global
    var = tot[:, 1:2] / n_global - mean * mean
    rstd = lax.rsqrt(var + EPS)

    return _normalize(
        x, mean, rstd, gamma.reshape(1, Nl), beta.reshape(1, Nl)
    )


# device time: 87497 ns/iter; 1.1243x vs baseline; 1.1243x over previous
import jax
import jax.numpy as jnp
from jax import lax
from jax.experimental import pallas as pl
from jax.experimental.pallas import tpu as pltpu

N_DEV = 4
EPS = 1e-5


def _partial_stats(x):
    M, Nl = x.shape
    RB = 1024
    nblk = M // RB
    groups = Nl // 128

    def body(x_ref, s_ref, q_ref):
        xb = x_ref[:, :].astype(jnp.float32)
        x3 = xb.reshape(RB, groups, 128)
        s = jnp.sum(x3, axis=1)
        q = jnp.sum(x3 * x3, axis=1)
        s_ref[:, :] = jnp.sum(s, axis=1, keepdims=True)
        q_ref[:, :] = jnp.sum(q, axis=1, keepdims=True)

    return pl.pallas_call(
        body,
        grid=(nblk,),
        in_specs=[pl.BlockSpec((RB, Nl), lambda b: (b, 0))],
        out_specs=[
            pl.BlockSpec((RB, 1), lambda b: (b, 0)),
            pl.BlockSpec((RB, 1), lambda b: (b, 0)),
        ],
        out_shape=[
            jax.ShapeDtypeStruct((M, 1), jnp.float32),
            jax.ShapeDtypeStruct((M, 1), jnp.float32),
        ],
    )(x)


def _allreduce_stats(stats):

    def body(st_ref, out_ref, comm_ref, send_sems, recv_sems):
        my = lax.axis_index("i")

        barrier_sem = pltpu.get_barrier_semaphore()
        for off in (1, 2, 3):
            pl.semaphore_signal(
                barrier_sem,
                inc=1,
                device_id=(lax.rem(my + off, N_DEV),),
                device_id_type=pl.DeviceIdType.MESH,
            )
        pl.semaphore_wait(barrier_sem, 3)

        sends = []
        for off in (1, 2, 3):
            tgt = lax.rem(my + off, N_DEV)
            r = pltpu.make_async_remote_copy(
                src_ref=st_ref,
                dst_ref=comm_ref.at[my],
                send_sem=send_sems.at[off],
                recv_sem=recv_sems.at[my],
                device_id=(tgt,),
                device_id_type=pl.DeviceIdType.MESH,
            )
            r.start()
            sends.append(r)

        acc = st_ref[:, :]
        for off in (1, 2, 3):
            src = lax.rem(my - off + N_DEV, N_DEV)
            recv = pltpu.make_async_remote_copy(
                src_ref=st_ref,
                dst_ref=comm_ref.at[src],
                send_sem=send_sems.at[0],
                recv_sem=recv_sems.at[src],
                device_id=(my,),
                device_id_type=pl.DeviceIdType.MESH,
            )
            recv.wait_recv()
            acc = acc + comm_ref[src]
        out_ref[:, :] = acc

        for r in sends:
            r.wait_send()

    return pl.pallas_call(
        body,
        out_shape=jax.ShapeDtypeStruct((128, 128), jnp.float32),
        in_specs=[pl.BlockSpec(memory_space=pltpu.VMEM)],
        out_specs=pl.BlockSpec(memory_space=pltpu.VMEM),
        scratch_shapes=[
            pltpu.VMEM((N_DEV, 128, 128), jnp.float32),
            pltpu.SemaphoreType.DMA((N_DEV,)),
            pltpu.SemaphoreType.DMA((N_DEV,)),
        ],
        compiler_params=pltpu.CompilerParams(collective_id=0),
    )(stats)


def _normalize(x, mean, rstd, gamma2, beta2):
    M, Nl = x.shape
    RB = 512
    nblk = M // RB

    def body(x_ref, m_ref, r_ref, g_ref, b_ref, o_ref):
        xb = x_ref[:, :].astype(jnp.float32)
        o_ref[:, :] = (
            (xb - m_ref[:, :]) * r_ref[:, :] * g_ref[:, :] + b_ref[:, :]
        ).astype(jnp.bfloat16)

    return pl.pallas_call(
        body,
        grid=(nblk,),
        in_specs=[
            pl.BlockSpec((RB, Nl), lambda b: (b, 0)),
            pl.BlockSpec((RB, 1), lambda b: (b, 0)),
            pl.BlockSpec((RB, 1), lambda b: (b, 0)),
            pl.BlockSpec((1, Nl), lambda b: (0, 0)),
            pl.BlockSpec((1, Nl), lambda b: (0, 0)),
        ],
        out_specs=pl.BlockSpec((RB, Nl), lambda b: (b, 0)),
        out_shape=jax.ShapeDtypeStruct((M, Nl), jnp.bfloat16),
    )(x, mean, rstd, gamma2, beta2)


def kernel(x, gamma, beta):
    M, Nl = x.shape
    n_global = Nl * N_DEV

    sums, sumsq = _partial_stats(x)
    stats = jnp.concatenate([sums, sumsq], axis=1).reshape(128, 128)

    red = _allreduce_stats(stats)

    tot = red.reshape(M, 2)
    mean = tot[:, 0:1] / n_global
    var = tot[:, 1:2] / n_global - mean * mean
    rstd = lax.rsqrt(var + EPS)

    return _normalize(
        x, mean, rstd, gamma.reshape(1, Nl), beta.reshape(1, Nl)
    )


# device time: 77544 ns/iter; 1.2686x vs baseline; 1.1284x over previous
import jax
import jax.numpy as jnp
from jax import lax
from jax.experimental import pallas as pl
from jax.experimental.pallas import tpu as pltpu

N_DEV = 4
EPS = 1e-5
RB = 512


def kernel(x, gamma, beta):
    M, Nl = x.shape
    nblk = M // RB
    n_global = Nl * N_DEV

    def body(x_ref, g_ref, b_ref, o_ref, xcache, stats, comm, mr,
             send_sems, recv_sems):
        g = pl.program_id(0)
        my = lax.axis_index("i")

        ones_row = jnp.ones((1, Nl), dtype=jnp.float32)

        @pl.when(g < nblk)
        def _phase0():
            xb = x_ref[:, :]
            xcache[pl.ds(g * RB, RB), :] = xb.astype(jnp.bfloat16)
            s_row = lax.dot_general(
                ones_row, xb, (((1,), (1,)), ((), ())),
                preferred_element_type=jnp.float32,
            )
            q_row = lax.dot_general(
                ones_row, xb * xb, (((1,), (1,)), ((), ())),
                preferred_element_type=jnp.float32,
            )
            stats[pl.ds(g, 1), :] = s_row
            stats[pl.ds(nblk + g, 1), :] = q_row

        @pl.when(g == nblk)
        def _allreduce():
            barrier_sem = pltpu.get_barrier_semaphore()
            for off in (1, 2, 3):
                pl.semaphore_signal(
                    barrier_sem, inc=1,
                    device_id=(lax.rem(my + off, N_DEV),),
                    device_id_type=pl.DeviceIdType.MESH,
                )
            pl.semaphore_wait(barrier_sem, 3)

            sends = []
            for off in (1, 2, 3):
                r = pltpu.make_async_remote_copy(
                    src_ref=stats,
                    dst_ref=comm.at[my],
                    send_sem=send_sems.at[off],
                    recv_sem=recv_sems.at[my],
                    device_id=(lax.rem(my + off, N_DEV),),
                    device_id_type=pl.DeviceIdType.MESH,
                )
                r.start()
                sends.append(r)

            acc = stats[:, :]
            for off in (1, 2, 3):
                src = lax.rem(my - off + N_DEV, N_DEV)
                recv = pltpu.make_async_remote_copy(
                    src_ref=stats,
                    dst_ref=comm.at[src],
                    send_sem=send_sems.at[0],
                    recv_sem=recv_sems.at[src],
                    device_id=(my,),
                    device_id_type=pl.DeviceIdType.MESH,
                )
                recv.wait_recv()
                acc = acc + comm[src]

            mean_rows = acc[:nblk, :] / n_global
            var_rows = acc[nblk:, :] / n_global - mean_rows * mean_rows
            rstd_rows = lax.rsqrt(var_rows + EPS)
            mr[pl.ds(0, nblk), :] = mean_rows
            mr[pl.ds(nblk, nblk), :] = rstd_rows

            for r in sends:
                r.wait_send()

        @pl.when(g >= nblk)
        def _phase1():
            b = g - nblk
            mr2 = jnp.concatenate(
                [mr[pl.ds(b, 1), :], mr[pl.ds(nblk + b, 1), :]], axis=0
            )
            eye = jnp.eye(RB, dtype=jnp.float32)
            cols = lax.dot_general(
                eye, mr2, (((1,), (1,)), ((), ())),
                preferred_element_type=jnp.float32,
            )
            m_col = cols[:, 0:1]
            r_col = cols[:, 1:2]
            xb = xcache[pl.ds(b * RB, RB), :].astype(jnp.float32)
            o_ref[:, :] = (
                (xb - m_col) * r_col * g_ref[:, :] + b_ref[:, :]
            ).astype(jnp.bfloat16)

    out = pl.pallas_call(
        body,
        grid=(2 * nblk,),
        in_specs=[
            pl.BlockSpec((RB, Nl), lambda g: (jnp.minimum(g, nblk - 1), 0)),
            pl.BlockSpec((1, Nl), lambda g: (0, 0)),
            pl.BlockSpec((1, Nl), lambda g: (0, 0)),
        ],
        out_specs=pl.BlockSpec((RB, Nl), lambda g: (jnp.maximum(g - nblk, 0), 0)),
        out_shape=jax.ShapeDtypeStruct((M, Nl), jnp.bfloat16),
        scratch_shapes=[
            pltpu.VMEM((M, Nl), jnp.bfloat16),
            pltpu.VMEM((2 * nblk, RB), jnp.float32),
            pltpu.VMEM((N_DEV, 2 * nblk, RB), jnp.float32),
            pltpu.VMEM((2 * nblk, RB), jnp.float32),
            pltpu.SemaphoreType.DMA((N_DEV,)),
            pltpu.SemaphoreType.DMA((N_DEV,)),
        ],
        compiler_params=pltpu.CompilerParams(
            collective_id=0,
            vmem_limit_bytes=100 * 1024 * 1024,
        ),
    )(x, gamma.reshape(1, Nl), beta.reshape(1, Nl))
    return out
